# baseline (speedup 1.0000x reference)
import functools

import jax
import jax.numpy as jnp
from jax.experimental import pallas as pl
from jax.experimental.pallas import tpu as pltpu

T, N, E, B = 12, 10000, 320000, 64
DIN, DH, HL = 84, 128, 100


def _head_body(seq_ref, wih0_ref, whh0_ref, bg0_ref, wih1_ref, whh1_ref,
               bg1_ref, fc1w_ref, fc1b_ref, fc2w_ref, fc2b_ref, fc3w_ref,
               fc3b_ref, g1_ref, be1_ref, g2_ref, be2_ref, out_ref,
               acc_ref):
    def make_step(wih_ref, whh_ref, bg_ref):
        def step(xt, h, c):
            gates = []
            for g in range(4):
                gg = (jnp.dot(xt, wih_ref[g], preferred_element_type=jnp.float32)
                      + jnp.dot(h, whh_ref[g], preferred_element_type=jnp.float32)
                      + bg_ref[g])
                gates.append(gg)
            i = jax.nn.sigmoid(gates[0])
            f = jax.nn.sigmoid(gates[1])
            gg = jnp.tanh(gates[2])
            o = jax.nn.sigmoid(gates[3])
            c = f * c + i * gg
            h = o * jnp.tanh(c)
            return h, c
        return step

    step0 = make_step(wih0_ref, whh0_ref, bg0_ref)
    step1 = make_step(wih1_ref, whh1_ref, bg1_ref)

    z = jnp.zeros((B, HL), jnp.float32)
    h0, c0, h1, c1 = z, z, z, z
    acc_ref[...] = jnp.zeros_like(acc_ref)
    for t in range(T):
        xt = seq_ref[t]
        h0, c0 = step0(xt, h0, c0)
        h1, c1 = step1(h0, h1, c1)
        acc_ref[...] += jnp.dot(h1, fc1w_ref[t],
                                preferred_element_type=jnp.float32)

    def bn(h, g, b):
        mu = jnp.mean(h, axis=0, keepdims=True)
        var = jnp.mean((h - mu) ** 2, axis=0, keepdims=True)
        return g * (h - mu) * jax.lax.rsqrt(var + 1e-5) + b

    h = acc_ref[...] + fc1b_ref[...]
    h = h * jax.nn.sigmoid(h)
    h = bn(h, g1_ref[...], be1_ref[...])
    h = jnp.dot(h, fc2w_ref[...], preferred_element_type=jnp.float32) + fc2b_ref[...]
    h = h * jax.nn.sigmoid(h)
    h = bn(h, g2_ref[...], be2_ref[...])
    h = jnp.dot(h, fc3w_ref[...], preferred_element_type=jnp.float32) + fc3b_ref[...]
    h = h - jnp.max(h, axis=1, keepdims=True)
    eh = jnp.exp(h)
    out_ref[...] = eh / jnp.sum(eh, axis=1, keepdims=True)


def _head(seq, W_ih0, W_hh0, b_ih0, b_hh0, W_ih1, W_hh1, b_ih1, b_hh1,
          fc1_w, fc1_b, fc2_w, fc2_b, fc3_w, fc3_b, g1, be1, g2, be2):
    wih0 = jnp.transpose(W_ih0.reshape(4, HL, 2 * DH), (0, 2, 1))
    whh0 = jnp.transpose(W_hh0.reshape(4, HL, HL), (0, 2, 1))
    bg0 = (b_ih0 + b_hh0).reshape(4, 1, HL)
    wih1 = jnp.transpose(W_ih1.reshape(4, HL, HL), (0, 2, 1))
    whh1 = jnp.transpose(W_hh1.reshape(4, HL, HL), (0, 2, 1))
    bg1 = (b_ih1 + b_hh1).reshape(4, 1, HL)
    fc1w = jnp.transpose(fc1_w.reshape(512, T, HL), (1, 2, 0))
    fc2w = fc2_w.T
    fc3w = fc3_w.T
    return pl.pallas_call(
        _head_body,
        out_shape=jax.ShapeDtypeStruct((B, 2), jnp.float32),
        scratch_shapes=[pltpu.VMEM((B, 512), jnp.float32)],
    )(seq, wih0, whh0, bg0, wih1, whh1, bg1, fc1w, fc1_b.reshape(1, 512),
      fc2w, fc2_b.reshape(1, 32), fc3w, fc3_b.reshape(1, 2),
      g1.reshape(1, 512), be1.reshape(1, 512), g2.reshape(1, 32),
      be2.reshape(1, 32))


def kernel(x, edge_index, edge_attr, batch, y, Wl1, bl1, Wr1, Wl2, bl2, Wr2,
           W_ih0, W_hh0, b_ih0, b_hh0, W_ih1, W_hh1, b_ih1, b_hh1, fc1_w,
           fc1_b, fc2_w, fc2_b, fc3_w, fc3_b, g1, be1, g2, be2):
    src, dst = edge_index[0], edge_index[1]
    deg = jnp.clip(jax.ops.segment_sum(jnp.ones((E,), x.dtype), dst,
                                       num_segments=N), 1.0)
    cnt = jnp.clip(jax.ops.segment_sum(jnp.ones((N,), x.dtype), batch,
                                       num_segments=B), 1.0)

    def sage(h, Wl, bl, Wr):
        aggr = jax.ops.segment_sum(h[src], dst, num_segments=N) / deg[:, None]
        return aggr @ Wl.T + bl + h @ Wr.T

    feats = []
    for t in range(T):
        h1 = sage(x[t], Wl1, bl1, Wr1)
        h2 = sage(h1, Wl2, bl2, Wr2)
        gmax = jax.ops.segment_max(h1, batch, num_segments=B)
        gmean = jax.ops.segment_sum(h2, batch, num_segments=B) / cnt[:, None]
        feats.append(jnp.concatenate([gmax, gmean], axis=1))
    seq = jnp.stack(feats)

    out = _head(seq, W_ih0, W_hh0, b_ih0, b_hh0, W_ih1, W_hh1, b_ih1, b_hh1,
                fc1_w, fc1_b, fc2_w, fc2_b, fc3_w, fc3_b, g1, be1, g2, be2)
    return (out, y)


# speedup vs baseline: 3.5436x; 3.5436x over previous
import functools

import jax
import jax.numpy as jnp
from jax.experimental import pallas as pl
from jax.experimental.pallas import tpu as pltpu
from jax.experimental.pallas import tpu_sc as plsc

T, N, E, B = 12, 10000, 320000, 64
DIN, DH, HL = 84, 128, 100
DP = 96
W = 80
NTILES = 32
EPT = E // NTILES
CH = EPT // W
RPT = 1000

_f32 = jnp.float32


def _make_sc_aggr(nrounds, F, acc_rounds):
    mesh = plsc.VectorSubcoreMesh(core_axis_name="c", subcore_axis_name="s")

    @functools.partial(
        pl.kernel,
        out_type=jax.ShapeDtypeStruct((2, nrounds, N, F), _f32),
        mesh=mesh,
        scratch_types=[
            pltpu.VMEM((CH, W), jnp.int32),
            pltpu.VMEM((CH, W), jnp.int32),
            pltpu.VMEM((W, F), _f32),
            pltpu.VMEM_SHARED((acc_rounds, N, F), _f32),
        ],
        compiler_params=pltpu.CompilerParams(use_tc_tiling_on_sc=False),
    )
    def sc_aggr(feats_ref, src_ref, dst_ref, zeros_ref, out_ref,
                src_v, dst_v, rows_v, acc):
        c = jax.lax.axis_index("c")
        s = jax.lax.axis_index("s")
        wid = s * 2 + c
        pltpu.sync_copy(src_ref.at[wid], src_v)
        pltpu.sync_copy(dst_ref.at[wid], dst_v)
        for r0 in range(0, nrounds, acc_rounds):
            @pl.when(s < N // RPT)
            def _():
                for j in range(acc_rounds):
                    pltpu.sync_copy(zeros_ref, acc.at[j, pl.ds(s * RPT, RPT)])
            plsc.subcore_barrier()

            @pl.loop(0, CH)
            def _(k):
                si = src_v.at[k]
                di = dst_v.at[k]
                for j in range(acc_rounds):
                    pltpu.sync_copy(feats_ref.at[r0 + j].at[si], rows_v)
                    pltpu.sync_copy(rows_v, acc.at[j].at[di], add=True)

            plsc.subcore_barrier()

            @pl.when(s < N // RPT)
            def _():
                for j in range(acc_rounds):
                    pltpu.sync_copy(acc.at[j, pl.ds(s * RPT, RPT)],
                                    out_ref.at[c, r0 + j, pl.ds(s * RPT, RPT)])
            plsc.subcore_barrier()

    return sc_aggr


NB1 = 1000


def _tc1_body(ad_ref, xp_ref, wl_ref, wr_ref, b_ref, out_ref):
    hp = jax.lax.Precision.HIGHEST
    m = jnp.dot(ad_ref[0], wl_ref[...], preferred_element_type=_f32,
                precision=hp)
    out_ref[0] = (m + b_ref[...]
                  + jnp.dot(xp_ref[0], wr_ref[...], preferred_element_type=_f32,
                            precision=hp))


def _tc1(adiv1, xp, wl, wr, bl):
    return pl.pallas_call(
        _tc1_body,
        grid=(T, N // NB1),
        in_specs=[
            pl.BlockSpec((1, NB1, DP), lambda t, n: (t, n, 0)),
            pl.BlockSpec((1, NB1, DP), lambda t, n: (t, n, 0)),
            pl.BlockSpec((DP, DH), lambda t, n: (0, 0)),
            pl.BlockSpec((DP, DH), lambda t, n: (0, 0)),
            pl.BlockSpec((1, DH), lambda t, n: (0, 0)),
        ],
        out_specs=pl.BlockSpec((1, NB1, DH), lambda t, n: (t, n, 0)),
        out_shape=jax.ShapeDtypeStruct((T, N, DH), _f32),
    )(adiv1, xp, wl, wr, bl)


NBG = 1000


def _gmax_body(h1_ref, bc_ref, out_ref):
    nb = pl.program_id(1)

    @pl.when(nb == 0)
    def _():
        out_ref[...] = jnp.full_like(out_ref, -jnp.inf)

    hb = h1_ref[0]
    bc = bc_ref[...]
    parts = []
    for b in range(B):
        sel = jnp.where(bc == b, hb, -jnp.inf)
        parts.append(jnp.max(sel, axis=0, keepdims=True))
    out_ref[0] = jnp.maximum(out_ref[0], jnp.concatenate(parts, axis=0))


def _gmax(h1, batch_col):
    return pl.pallas_call(
        _gmax_body,
        grid=(T, N // NBG),
        in_specs=[
            pl.BlockSpec((1, NBG, DH), lambda t, n: (t, n, 0)),
            pl.BlockSpec((NBG, 1), lambda t, n: (n, 0)),
        ],
        out_specs=pl.BlockSpec((1, B, DH), lambda t, n: (t, 0, 0)),
        out_shape=jax.ShapeDtypeStruct((T, B, DH), _f32),
    )(h1, batch_col)


def _tc2_body(ad2_ref, h1_ref, br_ref, wl_ref, wr_ref, b_ref,
              out_ref, cnto_ref, cnt_ref):
    t = pl.program_id(0)
    nb = pl.program_id(1)
    nbn = pl.num_programs(1)
    hp = jax.lax.Precision.HIGHEST
    h2 = (jnp.dot(ad2_ref[0], wl_ref[...], preferred_element_type=_f32,
                  precision=hp)
          + b_ref[...]
          + jnp.dot(h1_ref[0], wr_ref[...], preferred_element_type=_f32,
                    precision=hp))
    S = (jax.lax.broadcasted_iota(jnp.int32, (B, NB1), 0)
         == br_ref[0]).astype(_f32)

    @pl.when(nb == 0)
    def _():
        out_ref[...] = jnp.zeros_like(out_ref)
        cnt_ref[...] = jnp.zeros_like(cnt_ref)

    out_ref[0] += jnp.dot(S, h2, preferred_element_type=_f32,
                          precision=jax.lax.Precision.HIGHEST)
    cnt_ref[...] += jnp.sum(S, axis=1, keepdims=True)

    @pl.when((nb == nbn - 1) & (t == 0))
    def _():
        cnto_ref[...] = cnt_ref[...]


def _tc2(adiv2, h1, batch_row, wl, wr, bl):
    return pl.pallas_call(
        _tc2_body,
        grid=(T, N // NB1),
        in_specs=[
            pl.BlockSpec((1, NB1, DH), lambda t, n: (t, n, 0)),
            pl.BlockSpec((1, NB1, DH), lambda t, n: (t, n, 0)),
            pl.BlockSpec((1, 1, NB1), lambda t, n: (n, 0, 0)),
            pl.BlockSpec((DH, DH), lambda t, n: (0, 0)),
            pl.BlockSpec((DH, DH), lambda t, n: (0, 0)),
            pl.BlockSpec((1, DH), lambda t, n: (0, 0)),
        ],
        out_specs=[
            pl.BlockSpec((1, B, DH), lambda t, n: (t, 0, 0)),
            pl.BlockSpec((B, 1), lambda t, n: (0, 0)),
        ],
        out_shape=[
            jax.ShapeDtypeStruct((T, B, DH), _f32),
            jax.ShapeDtypeStruct((B, 1), _f32),
        ],
        scratch_shapes=[pltpu.VMEM((B, 1), _f32)],
    )(adiv2, h1, batch_row, wl, wr, bl)


def _head_body(gmax_ref, gmean_ref, wih0_ref, whh0_ref, bg0_ref, wih1_ref,
               whh1_ref, bg1_ref, fc1w_ref, fc1b_ref, fc2w_ref, fc2b_ref,
               fc3w_ref, fc3b_ref, g1_ref, be1_ref, g2_ref, be2_ref, out_ref,
               acc_ref):
    def make_step(wih_ref, whh_ref, bg_ref):
        def step(xt, h, c):
            gates = []
            for g in range(4):
                gg = (jnp.dot(xt, wih_ref[g], preferred_element_type=_f32)
                      + jnp.dot(h, whh_ref[g], preferred_element_type=_f32)
                      + bg_ref[g])
                gates.append(gg)
            i = jax.nn.sigmoid(gates[0])
            f = jax.nn.sigmoid(gates[1])
            gg = jnp.tanh(gates[2])
            o = jax.nn.sigmoid(gates[3])
            c = f * c + i * gg
            h = o * jnp.tanh(c)
            return h, c
        return step

    step0 = make_step(wih0_ref, whh0_ref, bg0_ref)
    step1 = make_step(wih1_ref, whh1_ref, bg1_ref)

    z = jnp.zeros((B, HL), _f32)
    h0, c0, h1, c1 = z, z, z, z
    acc_ref[...] = jnp.zeros_like(acc_ref)
    for t in range(T):
        xt = jnp.concatenate([gmax_ref[t], gmean_ref[t]], axis=-1)
        h0, c0 = step0(xt, h0, c0)
        h1, c1 = step1(h0, h1, c1)
        acc_ref[...] += jnp.dot(h1, fc1w_ref[t], preferred_element_type=_f32)

    def bn(h, g, b):
        mu = jnp.mean(h, axis=0, keepdims=True)
        var = jnp.mean((h - mu) ** 2, axis=0, keepdims=True)
        return g * (h - mu) * jax.lax.rsqrt(var + 1e-5) + b

    h = acc_ref[...] + fc1b_ref[...]
    h = h * jax.nn.sigmoid(h)
    h = bn(h, g1_ref[...], be1_ref[...])
    h = jnp.dot(h, fc2w_ref[...], preferred_element_type=_f32) + fc2b_ref[...]
    h = h * jax.nn.sigmoid(h)
    h = bn(h, g2_ref[...], be2_ref[...])
    h = jnp.dot(h, fc3w_ref[...], preferred_element_type=_f32) + fc3b_ref[...]
    h = h - jnp.max(h, axis=1, keepdims=True)
    eh = jnp.exp(h)
    out_ref[...] = eh / jnp.sum(eh, axis=1, keepdims=True)


def _head(gmax, gmean, W_ih0, W_hh0, b_ih0, b_hh0, W_ih1, W_hh1, b_ih1, b_hh1,
          fc1_w, fc1_b, fc2_w, fc2_b, fc3_w, fc3_b, g1, be1, g2, be2):
    wih0 = jnp.transpose(W_ih0.reshape(4, HL, 2 * DH), (0, 2, 1))
    whh0 = jnp.transpose(W_hh0.reshape(4, HL, HL), (0, 2, 1))
    bg0 = (b_ih0 + b_hh0).reshape(4, 1, HL)
    wih1 = jnp.transpose(W_ih1.reshape(4, HL, HL), (0, 2, 1))
    whh1 = jnp.transpose(W_hh1.reshape(4, HL, HL), (0, 2, 1))
    bg1 = (b_ih1 + b_hh1).reshape(4, 1, HL)
    fc1w = jnp.transpose(fc1_w.reshape(512, T, HL), (1, 2, 0))
    return pl.pallas_call(
        _head_body,
        out_shape=jax.ShapeDtypeStruct((B, 2), _f32),
        scratch_shapes=[pltpu.VMEM((B, 512), _f32)],
    )(gmax, gmean, wih0, whh0, bg0, wih1, whh1, bg1, fc1w,
      fc1_b.reshape(1, 512), fc2_w.T, fc2_b.reshape(1, 32), fc3_w.T,
      fc3_b.reshape(1, 2), g1.reshape(1, 512), be1.reshape(1, 512),
      g2.reshape(1, 32), be2.reshape(1, 32))


def kernel(x, edge_index, edge_attr, batch, y, Wl1, bl1, Wr1, Wl2, bl2, Wr2,
           W_ih0, W_hh0, b_ih0, b_hh0, W_ih1, W_hh1, b_ih1, b_hh1, fc1_w,
           fc1_b, fc2_w, fc2_b, fc3_w, fc3_b, g1, be1, g2, be2):
    xp = jnp.pad(x, ((0, 0), (0, 0), (0, DP - DIN)))
    xp = xp.at[0, :, DIN].set(1.0)
    src2d = edge_index[0].reshape(NTILES, CH, W)
    dst2d = edge_index[1].reshape(NTILES, CH, W)
    zeros1 = jnp.zeros((RPT, DP), _f32)
    zeros2 = jnp.zeros((RPT, DH), _f32)

    wl1 = jnp.pad(Wl1.T, ((0, DP - DIN), (0, 0)))
    wr1 = jnp.pad(Wr1.T, ((0, DP - DIN), (0, 0)))
    bl1r = bl1.reshape(1, DH)
    wl2 = Wl2.T
    wr2 = Wr2.T
    bl2r = bl2.reshape(1, DH)
    batch_col = batch.reshape(N, 1)
    batch_row = batch.reshape(N // NB1, 1, NB1)

    sc1 = _make_sc_aggr(T, DP, acc_rounds=1)
    sc2 = _make_sc_aggr(T, DH, acc_rounds=1)

    aggr1 = sc1(xp, src2d, dst2d, zeros1)
    asum1 = aggr1[0] + aggr1[1]
    deg = jnp.clip(asum1[0, :, DIN:DIN + 1], 1.0, None)
    adiv1 = asum1 / deg
    h1 = _tc1(adiv1, xp, wl1, wr1, bl1r)
    gmax = _gmax(h1, batch_col)
    aggr2 = sc2(h1, src2d, dst2d, zeros2)
    adiv2 = (aggr2[0] + aggr2[1]) / deg
    gsum, cntb = _tc2(adiv2, h1, batch_row, wl2, wr2, bl2r)
    gmean = gsum / jnp.clip(cntb, 1.0, None)

    out = _head(gmax, gmean, W_ih0, W_hh0, b_ih0, b_hh0, W_ih1, W_hh1, b_ih1,
                b_hh1, fc1_w, fc1_b, fc2_w, fc2_b, fc3_w, fc3_b, g1, be1, g2,
                be2)
    return (out, y)


# speedup vs baseline: 5.6651x; 1.5987x over previous
import functools

import jax
import jax.numpy as jnp
from jax.experimental import pallas as pl
from jax.experimental.pallas import tpu as pltpu
from jax.experimental.pallas import tpu_sc as plsc

T, N, E, B = 12, 10000, 320000, 64
DIN, DH, HL = 84, 128, 100
DP = 96
W = 100
NTILES = 32
EPT = E // NTILES
CH = EPT // W
RPT = 1000

_f32 = jnp.float32


def _make_sc_aggr(nrounds, F):
    mesh = plsc.VectorSubcoreMesh(core_axis_name="c", subcore_axis_name="s")

    @functools.partial(
        pl.kernel,
        out_type=jax.ShapeDtypeStruct((2, nrounds, N, F), _f32),
        mesh=mesh,
        scratch_types=[
            pltpu.VMEM((CH, W), jnp.int32),
            pltpu.VMEM((CH, W), jnp.int32),
            pltpu.VMEM((W, F), _f32),
            pltpu.VMEM((W, F), _f32),
            pltpu.VMEM_SHARED((N, F), _f32),
            pltpu.SemaphoreType.DMA,
            pltpu.SemaphoreType.DMA,
        ],
        compiler_params=pltpu.CompilerParams(use_tc_tiling_on_sc=False),
    )
    def sc_aggr(feats_ref, src_ref, dst_ref, zeros_ref, out_ref,
                src_v, dst_v, rows0, rows1, acc, sem0, sem1):
        c = jax.lax.axis_index("c")
        s = jax.lax.axis_index("s")
        wid = s * 2 + c
        pltpu.sync_copy(src_ref.at[wid], src_v)
        pltpu.sync_copy(dst_ref.at[wid], dst_v)
        for r in range(nrounds):
            @pl.when(s < N // RPT)
            def _():
                pltpu.sync_copy(zeros_ref, acc.at[pl.ds(s * RPT, RPT)])
            plsc.subcore_barrier()

            fr = feats_ref.at[r]
            pltpu.async_copy(fr.at[src_v.at[0]], rows0, sem0)
            pltpu.async_copy(fr.at[src_v.at[1]], rows1, sem1)

            @pl.loop(0, CH, step=2)
            def _(k):
                pltpu.make_async_copy(fr.at[src_v.at[0]], rows0, sem0).wait()
                pltpu.sync_copy(rows0, acc.at[dst_v.at[k]], add=True)

                @pl.when(k + 2 < CH)
                def _():
                    pltpu.async_copy(fr.at[src_v.at[k + 2]], rows0, sem0)

                pltpu.make_async_copy(fr.at[src_v.at[1]], rows1, sem1).wait()
                pltpu.sync_copy(rows1, acc.at[dst_v.at[k + 1]], add=True)

                @pl.when(k + 3 < CH)
                def _():
                    pltpu.async_copy(fr.at[src_v.at[k + 3]], rows1, sem1)

            plsc.subcore_barrier()

            @pl.when(s < N // RPT)
            def _():
                pltpu.sync_copy(acc.at[pl.ds(s * RPT, RPT)],
                                out_ref.at[c, r, pl.ds(s * RPT, RPT)])
            plsc.subcore_barrier()

    return sc_aggr


NB1 = 1000


def _tc1_body(ad_ref, xp_ref, wl_ref, wr_ref, b_ref, out_ref):
    hp = jax.lax.Precision.HIGHEST
    m = jnp.dot(ad_ref[0], wl_ref[...], preferred_element_type=_f32,
                precision=hp)
    out_ref[0] = (m + b_ref[...]
                  + jnp.dot(xp_ref[0], wr_ref[...], preferred_element_type=_f32,
                            precision=hp))


def _tc1(adiv1, xp, wl, wr, bl):
    return pl.pallas_call(
        _tc1_body,
        grid=(T, N // NB1),
        in_specs=[
            pl.BlockSpec((1, NB1, DP), lambda t, n: (t, n, 0)),
            pl.BlockSpec((1, NB1, DP), lambda t, n: (t, n, 0)),
            pl.BlockSpec((DP, DH), lambda t, n: (0, 0)),
            pl.BlockSpec((DP, DH), lambda t, n: (0, 0)),
            pl.BlockSpec((1, DH), lambda t, n: (0, 0)),
        ],
        out_specs=pl.BlockSpec((1, NB1, DH), lambda t, n: (t, n, 0)),
        out_shape=jax.ShapeDtypeStruct((T, N, DH), _f32),
    )(adiv1, xp, wl, wr, bl)


NBG = 1000


def _gmax_body(h1_ref, bc_ref, out_ref):
    nb = pl.program_id(1)

    @pl.when(nb == 0)
    def _():
        out_ref[...] = jnp.full_like(out_ref, -jnp.inf)

    hb = h1_ref[0]
    bc = bc_ref[...]
    parts = []
    for b in range(B):
        sel = jnp.where(bc == b, hb, -jnp.inf)
        parts.append(jnp.max(sel, axis=0, keepdims=True))
    out_ref[0] = jnp.maximum(out_ref[0], jnp.concatenate(parts, axis=0))


def _gmax(h1, batch_col):
    return pl.pallas_call(
        _gmax_body,
        grid=(T, N // NBG),
        in_specs=[
            pl.BlockSpec((1, NBG, DH), lambda t, n: (t, n, 0)),
            pl.BlockSpec((NBG, 1), lambda t, n: (n, 0)),
        ],
        out_specs=pl.BlockSpec((1, B, DH), lambda t, n: (t, 0, 0)),
        out_shape=jax.ShapeDtypeStruct((T, B, DH), _f32),
    )(h1, batch_col)


def _tc2_body(ad2_ref, h1_ref, br_ref, wl_ref, wr_ref, b_ref,
              out_ref, cnto_ref, cnt_ref):
    t = pl.program_id(0)
    nb = pl.program_id(1)
    nbn = pl.num_programs(1)
    hp = jax.lax.Precision.HIGHEST
    h2 = (jnp.dot(ad2_ref[0], wl_ref[...], preferred_element_type=_f32,
                  precision=hp)
          + b_ref[...]
          + jnp.dot(h1_ref[0], wr_ref[...], preferred_element_type=_f32,
                    precision=hp))
    S = (jax.lax.broadcasted_iota(jnp.int32, (B, NB1), 0)
         == br_ref[0]).astype(_f32)

    @pl.when(nb == 0)
    def _():
        out_ref[...] = jnp.zeros_like(out_ref)
        cnt_ref[...] = jnp.zeros_like(cnt_ref)

    out_ref[0] += jnp.dot(S, h2, preferred_element_type=_f32,
                          precision=jax.lax.Precision.HIGHEST)
    cnt_ref[...] += jnp.sum(S, axis=1, keepdims=True)

    @pl.when((nb == nbn - 1) & (t == 0))
    def _():
        cnto_ref[...] = cnt_ref[...]


def _tc2(adiv2, h1, batch_row, wl, wr, bl):
    return pl.pallas_call(
        _tc2_body,
        grid=(T, N // NB1),
        in_specs=[
            pl.BlockSpec((1, NB1, DH), lambda t, n: (t, n, 0)),
            pl.BlockSpec((1, NB1, DH), lambda t, n: (t, n, 0)),
            pl.BlockSpec((1, 1, NB1), lambda t, n: (n, 0, 0)),
            pl.BlockSpec((DH, DH), lambda t, n: (0, 0)),
            pl.BlockSpec((DH, DH), lambda t, n: (0, 0)),
            pl.BlockSpec((1, DH), lambda t, n: (0, 0)),
        ],
        out_specs=[
            pl.BlockSpec((1, B, DH), lambda t, n: (t, 0, 0)),
            pl.BlockSpec((B, 1), lambda t, n: (0, 0)),
        ],
        out_shape=[
            jax.ShapeDtypeStruct((T, B, DH), _f32),
            jax.ShapeDtypeStruct((B, 1), _f32),
        ],
        scratch_shapes=[pltpu.VMEM((B, 1), _f32)],
    )(adiv2, h1, batch_row, wl, wr, bl)


def _head_body(gmax_ref, gmean_ref, wih0_ref, whh0_ref, bg0_ref, wih1_ref,
               whh1_ref, bg1_ref, fc1w_ref, fc1b_ref, fc2w_ref, fc2b_ref,
               fc3w_ref, fc3b_ref, g1_ref, be1_ref, g2_ref, be2_ref, out_ref,
               acc_ref):
    def make_step(wih_ref, whh_ref, bg_ref):
        def step(xt, h, c):
            gates = []
            for g in range(4):
                gg = (jnp.dot(xt, wih_ref[g], preferred_element_type=_f32)
                      + jnp.dot(h, whh_ref[g], preferred_element_type=_f32)
                      + bg_ref[g])
                gates.append(gg)
            i = jax.nn.sigmoid(gates[0])
            f = jax.nn.sigmoid(gates[1])
            gg = jnp.tanh(gates[2])
            o = jax.nn.sigmoid(gates[3])
            c = f * c + i * gg
            h = o * jnp.tanh(c)
            return h, c
        return step

    step0 = make_step(wih0_ref, whh0_ref, bg0_ref)
    step1 = make_step(wih1_ref, whh1_ref, bg1_ref)

    z = jnp.zeros((B, HL), _f32)
    h0, c0, h1, c1 = z, z, z, z
    acc_ref[...] = jnp.zeros_like(acc_ref)
    for t in range(T):
        xt = jnp.concatenate([gmax_ref[t], gmean_ref[t]], axis=-1)
        h0, c0 = step0(xt, h0, c0)
        h1, c1 = step1(h0, h1, c1)
        acc_ref[...] += jnp.dot(h1, fc1w_ref[t], preferred_element_type=_f32)

    def bn(h, g, b):
        mu = jnp.mean(h, axis=0, keepdims=True)
        var = jnp.mean((h - mu) ** 2, axis=0, keepdims=True)
        return g * (h - mu) * jax.lax.rsqrt(var + 1e-5) + b

    h = acc_ref[...] + fc1b_ref[...]
    h = h * jax.nn.sigmoid(h)
    h = bn(h, g1_ref[...], be1_ref[...])
    h = jnp.dot(h, fc2w_ref[...], preferred_element_type=_f32) + fc2b_ref[...]
    h = h * jax.nn.sigmoid(h)
    h = bn(h, g2_ref[...], be2_ref[...])
    h = jnp.dot(h, fc3w_ref[...], preferred_element_type=_f32) + fc3b_ref[...]
    h = h - jnp.max(h, axis=1, keepdims=True)
    eh = jnp.exp(h)
    out_ref[...] = eh / jnp.sum(eh, axis=1, keepdims=True)


def _head(gmax, gmean, W_ih0, W_hh0, b_ih0, b_hh0, W_ih1, W_hh1, b_ih1, b_hh1,
          fc1_w, fc1_b, fc2_w, fc2_b, fc3_w, fc3_b, g1, be1, g2, be2):
    wih0 = jnp.transpose(W_ih0.reshape(4, HL, 2 * DH), (0, 2, 1))
    whh0 = jnp.transpose(W_hh0.reshape(4, HL, HL), (0, 2, 1))
    bg0 = (b_ih0 + b_hh0).reshape(4, 1, HL)
    wih1 = jnp.transpose(W_ih1.reshape(4, HL, HL), (0, 2, 1))
    whh1 = jnp.transpose(W_hh1.reshape(4, HL, HL), (0, 2, 1))
    bg1 = (b_ih1 + b_hh1).reshape(4, 1, HL)
    fc1w = jnp.transpose(fc1_w.reshape(512, T, HL), (1, 2, 0))
    return pl.pallas_call(
        _head_body,
        out_shape=jax.ShapeDtypeStruct((B, 2), _f32),
        scratch_shapes=[pltpu.VMEM((B, 512), _f32)],
    )(gmax, gmean, wih0, whh0, bg0, wih1, whh1, bg1, fc1w,
      fc1_b.reshape(1, 512), fc2_w.T, fc2_b.reshape(1, 32), fc3_w.T,
      fc3_b.reshape(1, 2), g1.reshape(1, 512), be1.reshape(1, 512),
      g2.reshape(1, 32), be2.reshape(1, 32))


def kernel(x, edge_index, edge_attr, batch, y, Wl1, bl1, Wr1, Wl2, bl2, Wr2,
           W_ih0, W_hh0, b_ih0, b_hh0, W_ih1, W_hh1, b_ih1, b_hh1, fc1_w,
           fc1_b, fc2_w, fc2_b, fc3_w, fc3_b, g1, be1, g2, be2):
    xp = jnp.pad(x, ((0, 0), (0, 0), (0, DP - DIN)))
    xp = xp.at[0, :, DIN].set(1.0)
    src2d = edge_index[0].reshape(NTILES, CH, W)
    dst2d = edge_index[1].reshape(NTILES, CH, W)
    zeros1 = jnp.zeros((RPT, DP), _f32)
    zeros2 = jnp.zeros((RPT, DH), _f32)

    wl1 = jnp.pad(Wl1.T, ((0, DP - DIN), (0, 0)))
    wr1 = jnp.pad(Wr1.T, ((0, DP - DIN), (0, 0)))
    bl1r = bl1.reshape(1, DH)
    wl2 = Wl2.T
    wr2 = Wr2.T
    bl2r = bl2.reshape(1, DH)
    batch_col = batch.reshape(N, 1)
    batch_row = batch.reshape(N // NB1, 1, NB1)

    sc1 = _make_sc_aggr(T, DP)
    sc2 = _make_sc_aggr(T, DH)

    aggr1 = sc1(xp, src2d, dst2d, zeros1)
    asum1 = aggr1[0] + aggr1[1]
    deg = jnp.clip(asum1[0, :, DIN:DIN + 1], 1.0, None)
    adiv1 = asum1 / deg
    h1 = _tc1(adiv1, xp, wl1, wr1, bl1r)
    gmax = _gmax(h1, batch_col)
    aggr2 = sc2(h1, src2d, dst2d, zeros2)
    adiv2 = (aggr2[0] + aggr2[1]) / deg
    gsum, cntb = _tc2(adiv2, h1, batch_row, wl2, wr2, bl2r)
    gmean = gsum / jnp.clip(cntb, 1.0, None)

    out = _head(gmax, gmean, W_ih0, W_hh0, b_ih0, b_hh0, W_ih1, W_hh1, b_ih1,
                b_hh1, fc1_w, fc1_b, fc2_w, fc2_b, fc3_w, fc3_b, g1, be1, g2,
                be2)
    return (out, y)


# speedup vs baseline: 5.6987x; 1.0059x over previous
import functools

import jax
import jax.numpy as jnp
from jax.experimental import pallas as pl
from jax.experimental.pallas import tpu as pltpu
from jax.experimental.pallas import tpu_sc as plsc

T, N, E, B = 12, 10000, 320000, 64
DIN, DH, HL = 84, 128, 100
DP = 96
NTILES = 32
EPT = E // NTILES
RPT = 1000

_f32 = jnp.float32


def _make_sc_aggr(nrounds, F, W):
    CH = EPT // W
    mesh = plsc.VectorSubcoreMesh(core_axis_name="c", subcore_axis_name="s")

    @functools.partial(
        pl.kernel,
        out_type=jax.ShapeDtypeStruct((2, nrounds, N, F), _f32),
        mesh=mesh,
        scratch_types=[
            pltpu.VMEM((CH, W), jnp.int32),
            pltpu.VMEM((CH, W), jnp.int32),
            pltpu.VMEM((W, F), _f32),
            pltpu.VMEM((W, F), _f32),
            pltpu.VMEM((W, F), _f32),
            pltpu.VMEM((W, F), _f32),
            pltpu.VMEM_SHARED((N, F), _f32),
            pltpu.SemaphoreType.DMA,
            pltpu.SemaphoreType.DMA,
            pltpu.SemaphoreType.DMA,
            pltpu.SemaphoreType.DMA,
            pltpu.SemaphoreType.DMA,
            pltpu.SemaphoreType.DMA,
            pltpu.SemaphoreType.DMA,
            pltpu.SemaphoreType.DMA,
        ],
        compiler_params=pltpu.CompilerParams(use_tc_tiling_on_sc=False),
    )
    def sc_aggr(feats_ref, src_ref, dst_ref, zeros_ref, out_ref,
                src_v, dst_v, rows0, rows1, rows2, rows3, acc,
                sg0, sg1, sg2, sg3, ss0, ss1, ss2, ss3):
        bufs = (rows0, rows1, rows2, rows3)
        sg = (sg0, sg1, sg2, sg3)
        ss = (ss0, ss1, ss2, ss3)
        c = jax.lax.axis_index("c")
        s = jax.lax.axis_index("s")
        wid = s * 2 + c
        pltpu.sync_copy(src_ref.at[wid], src_v)
        pltpu.sync_copy(dst_ref.at[wid], dst_v)
        for r in range(nrounds):
            @pl.when(s < N // RPT)
            def _():
                pltpu.sync_copy(zeros_ref, acc.at[pl.ds(s * RPT, RPT)])
            plsc.subcore_barrier()

            fr = feats_ref.at[r]
            for i in range(4):
                pltpu.async_copy(fr.at[src_v.at[i]], bufs[i], sg[i])

            @pl.loop(0, CH, step=4)
            def _(k):
                for i in range(4):
                    pltpu.make_async_copy(fr.at[src_v.at[0]], bufs[i],
                                          sg[i]).wait()
                    pltpu.async_copy(bufs[i], acc.at[dst_v.at[k + i]],
                                     ss[i], add=True)
                for i in range(4):
                    @pl.when(k + 4 + i < CH)
                    def _(i=i):
                        pltpu.make_async_copy(bufs[i], acc.at[dst_v.at[0]],
                                              ss[i]).wait()
                        pltpu.async_copy(fr.at[src_v.at[k + 4 + i]],
                                         bufs[i], sg[i])

            for i in range(4):
                pltpu.make_async_copy(bufs[i], acc.at[dst_v.at[0]],
                                      ss[i]).wait()
            plsc.subcore_barrier()

            @pl.when(s < N // RPT)
            def _():
                pltpu.sync_copy(acc.at[pl.ds(s * RPT, RPT)],
                                out_ref.at[c, r, pl.ds(s * RPT, RPT)])
            plsc.subcore_barrier()

    return sc_aggr


NB1 = 1000


def _tc1_body(ad_ref, xp_ref, wl_ref, wr_ref, b_ref, out_ref):
    hp = jax.lax.Precision.HIGHEST
    m = jnp.dot(ad_ref[0], wl_ref[...], preferred_element_type=_f32,
                precision=hp)
    out_ref[0] = (m + b_ref[...]
                  + jnp.dot(xp_ref[0], wr_ref[...], preferred_element_type=_f32,
                            precision=hp))


def _tc1(adiv1, xp, wl, wr, bl):
    return pl.pallas_call(
        _tc1_body,
        grid=(T, N // NB1),
        in_specs=[
            pl.BlockSpec((1, NB1, DP), lambda t, n: (t, n, 0)),
            pl.BlockSpec((1, NB1, DP), lambda t, n: (t, n, 0)),
            pl.BlockSpec((DP, DH), lambda t, n: (0, 0)),
            pl.BlockSpec((DP, DH), lambda t, n: (0, 0)),
            pl.BlockSpec((1, DH), lambda t, n: (0, 0)),
        ],
        out_specs=pl.BlockSpec((1, NB1, DH), lambda t, n: (t, n, 0)),
        out_shape=jax.ShapeDtypeStruct((T, N, DH), _f32),
    )(adiv1, xp, wl, wr, bl)


NBG = 1000


def _gmax_body(h1_ref, bc_ref, out_ref):
    nb = pl.program_id(1)

    @pl.when(nb == 0)
    def _():
        out_ref[...] = jnp.full_like(out_ref, -jnp.inf)

    hb = h1_ref[0]
    bc = bc_ref[...]
    parts = []
    for b in range(B):
        sel = jnp.where(bc == b, hb, -jnp.inf)
        parts.append(jnp.max(sel, axis=0, keepdims=True))
    out_ref[0] = jnp.maximum(out_ref[0], jnp.concatenate(parts, axis=0))


def _gmax(h1, batch_col):
    return pl.pallas_call(
        _gmax_body,
        grid=(T, N // NBG),
        in_specs=[
            pl.BlockSpec((1, NBG, DH), lambda t, n: (t, n, 0)),
            pl.BlockSpec((NBG, 1), lambda t, n: (n, 0)),
        ],
        out_specs=pl.BlockSpec((1, B, DH), lambda t, n: (t, 0, 0)),
        out_shape=jax.ShapeDtypeStruct((T, B, DH), _f32),
    )(h1, batch_col)


def _tc2_body(ad2_ref, h1_ref, br_ref, wl_ref, wr_ref, b_ref,
              out_ref, cnto_ref, cnt_ref):
    t = pl.program_id(0)
    nb = pl.program_id(1)
    nbn = pl.num_programs(1)
    hp = jax.lax.Precision.HIGHEST
    h2 = (jnp.dot(ad2_ref[0], wl_ref[...], preferred_element_type=_f32,
                  precision=hp)
          + b_ref[...]
          + jnp.dot(h1_ref[0], wr_ref[...], preferred_element_type=_f32,
                    precision=hp))
    S = (jax.lax.broadcasted_iota(jnp.int32, (B, NB1), 0)
         == br_ref[0]).astype(_f32)

    @pl.when(nb == 0)
    def _():
        out_ref[...] = jnp.zeros_like(out_ref)
        cnt_ref[...] = jnp.zeros_like(cnt_ref)

    out_ref[0] += jnp.dot(S, h2, preferred_element_type=_f32,
                          precision=jax.lax.Precision.HIGHEST)
    cnt_ref[...] += jnp.sum(S, axis=1, keepdims=True)

    @pl.when((nb == nbn - 1) & (t == 0))
    def _():
        cnto_ref[...] = cnt_ref[...]


def _tc2(adiv2, h1, batch_row, wl, wr, bl):
    return pl.pallas_call(
        _tc2_body,
        grid=(T, N // NB1),
        in_specs=[
            pl.BlockSpec((1, NB1, DH), lambda t, n: (t, n, 0)),
            pl.BlockSpec((1, NB1, DH), lambda t, n: (t, n, 0)),
            pl.BlockSpec((1, 1, NB1), lambda t, n: (n, 0, 0)),
            pl.BlockSpec((DH, DH), lambda t, n: (0, 0)),
            pl.BlockSpec((DH, DH), lambda t, n: (0, 0)),
            pl.BlockSpec((1, DH), lambda t, n: (0, 0)),
        ],
        out_specs=[
            pl.BlockSpec((1, B, DH), lambda t, n: (t, 0, 0)),
            pl.BlockSpec((B, 1), lambda t, n: (0, 0)),
        ],
        out_shape=[
            jax.ShapeDtypeStruct((T, B, DH), _f32),
            jax.ShapeDtypeStruct((B, 1), _f32),
        ],
        scratch_shapes=[pltpu.VMEM((B, 1), _f32)],
    )(adiv2, h1, batch_row, wl, wr, bl)


def _head_body(gmax_ref, gmean_ref, wih0_ref, whh0_ref, bg0_ref, wih1_ref,
               whh1_ref, bg1_ref, fc1w_ref, fc1b_ref, fc2w_ref, fc2b_ref,
               fc3w_ref, fc3b_ref, g1_ref, be1_ref, g2_ref, be2_ref, out_ref,
               acc_ref):
    def make_step(wih_ref, whh_ref, bg_ref):
        def step(xt, h, c):
            gates = []
            for g in range(4):
                gg = (jnp.dot(xt, wih_ref[g], preferred_element_type=_f32)
                      + jnp.dot(h, whh_ref[g], preferred_element_type=_f32)
                      + bg_ref[g])
                gates.append(gg)
            i = jax.nn.sigmoid(gates[0])
            f = jax.nn.sigmoid(gates[1])
            gg = jnp.tanh(gates[2])
            o = jax.nn.sigmoid(gates[3])
            c = f * c + i * gg
            h = o * jnp.tanh(c)
            return h, c
        return step

    step0 = make_step(wih0_ref, whh0_ref, bg0_ref)
    step1 = make_step(wih1_ref, whh1_ref, bg1_ref)

    z = jnp.zeros((B, HL), _f32)
    h0, c0, h1, c1 = z, z, z, z
    acc_ref[...] = jnp.zeros_like(acc_ref)
    for t in range(T):
        xt = jnp.concatenate([gmax_ref[t], gmean_ref[t]], axis=-1)
        h0, c0 = step0(xt, h0, c0)
        h1, c1 = step1(h0, h1, c1)
        acc_ref[...] += jnp.dot(h1, fc1w_ref[t], preferred_element_type=_f32)

    def bn(h, g, b):
        mu = jnp.mean(h, axis=0, keepdims=True)
        var = jnp.mean((h - mu) ** 2, axis=0, keepdims=True)
        return g * (h - mu) * jax.lax.rsqrt(var + 1e-5) + b

    h = acc_ref[...] + fc1b_ref[...]
    h = h * jax.nn.sigmoid(h)
    h = bn(h, g1_ref[...], be1_ref[...])
    h = jnp.dot(h, fc2w_ref[...], preferred_element_type=_f32) + fc2b_ref[...]
    h = h * jax.nn.sigmoid(h)
    h = bn(h, g2_ref[...], be2_ref[...])
    h = jnp.dot(h, fc3w_ref[...], preferred_element_type=_f32) + fc3b_ref[...]
    h = h - jnp.max(h, axis=1, keepdims=True)
    eh = jnp.exp(h)
    out_ref[...] = eh / jnp.sum(eh, axis=1, keepdims=True)


def _head(gmax, gmean, W_ih0, W_hh0, b_ih0, b_hh0, W_ih1, W_hh1, b_ih1, b_hh1,
          fc1_w, fc1_b, fc2_w, fc2_b, fc3_w, fc3_b, g1, be1, g2, be2):
    wih0 = jnp.transpose(W_ih0.reshape(4, HL, 2 * DH), (0, 2, 1))
    whh0 = jnp.transpose(W_hh0.reshape(4, HL, HL), (0, 2, 1))
    bg0 = (b_ih0 + b_hh0).reshape(4, 1, HL)
    wih1 = jnp.transpose(W_ih1.reshape(4, HL, HL), (0, 2, 1))
    whh1 = jnp.transpose(W_hh1.reshape(4, HL, HL), (0, 2, 1))
    bg1 = (b_ih1 + b_hh1).reshape(4, 1, HL)
    fc1w = jnp.transpose(fc1_w.reshape(512, T, HL), (1, 2, 0))
    return pl.pallas_call(
        _head_body,
        out_shape=jax.ShapeDtypeStruct((B, 2), _f32),
        scratch_shapes=[pltpu.VMEM((B, 512), _f32)],
    )(gmax, gmean, wih0, whh0, bg0, wih1, whh1, bg1, fc1w,
      fc1_b.reshape(1, 512), fc2_w.T, fc2_b.reshape(1, 32), fc3_w.T,
      fc3_b.reshape(1, 2), g1.reshape(1, 512), be1.reshape(1, 512),
      g2.reshape(1, 32), be2.reshape(1, 32))


def kernel(x, edge_index, edge_attr, batch, y, Wl1, bl1, Wr1, Wl2, bl2, Wr2,
           W_ih0, W_hh0, b_ih0, b_hh0, W_ih1, W_hh1, b_ih1, b_hh1, fc1_w,
           fc1_b, fc2_w, fc2_b, fc3_w, fc3_b, g1, be1, g2, be2):
    xp = jnp.pad(x, ((0, 0), (0, 0), (0, DP - DIN)))
    xp = xp.at[0, :, DIN].set(1.0)
    src2d_1 = edge_index[0].reshape(NTILES, 100, 100)
    dst2d_1 = edge_index[1].reshape(NTILES, 100, 100)
    src2d_2 = edge_index[0].reshape(NTILES, 200, 50)
    dst2d_2 = edge_index[1].reshape(NTILES, 200, 50)
    zeros1 = jnp.zeros((RPT, DP), _f32)
    zeros2 = jnp.zeros((RPT, DH), _f32)

    wl1 = jnp.pad(Wl1.T, ((0, DP - DIN), (0, 0)))
    wr1 = jnp.pad(Wr1.T, ((0, DP - DIN), (0, 0)))
    bl1r = bl1.reshape(1, DH)
    wl2 = Wl2.T
    wr2 = Wr2.T
    bl2r = bl2.reshape(1, DH)
    batch_col = batch.reshape(N, 1)
    batch_row = batch.reshape(N // NB1, 1, NB1)

    sc1 = _make_sc_aggr(T, DP, 100)
    sc2 = _make_sc_aggr(T, DH, 50)

    aggr1 = sc1(xp, src2d_1, dst2d_1, zeros1)
    asum1 = aggr1[0] + aggr1[1]
    deg = jnp.clip(asum1[0, :, DIN:DIN + 1], 1.0, None)
    adiv1 = asum1 / deg
    h1 = _tc1(adiv1, xp, wl1, wr1, bl1r)
    gmax = _gmax(h1, batch_col)
    aggr2 = sc2(h1, src2d_2, dst2d_2, zeros2)
    adiv2 = (aggr2[0] + aggr2[1]) / deg
    gsum, cntb = _tc2(adiv2, h1, batch_row, wl2, wr2, bl2r)
    gmean = gsum / jnp.clip(cntb, 1.0, None)

    out = _head(gmax, gmean, W_ih0, W_hh0, b_ih0, b_hh0, W_ih1, W_hh1, b_ih1,
                b_hh1, fc1_w, fc1_b, fc2_w, fc2_b, fc3_w, fc3_b, g1, be1, g2,
                be2)
    return (out, y)
